# baseline (speedup 1.0000x reference)
import functools

import jax
import jax.numpy as jnp
from jax import lax
from jax.experimental import pallas as pl
from jax.experimental.pallas import tpu as pltpu
from jax.experimental.pallas import tpu_sc as plsc

HIDDEN = 768
B_TOTAL = 8192
B_HALF = B_TOTAL // 2
NS = 16
B_PER_W = B_HALF // NS
CHUNK = 32
NCHUNK = B_PER_W // CHUNK
LANES = 16
COLS = HIDDEN // LANES


def _emb_body(widx_hbm, pidx_hbm, word_hbm, pos_hbm, out_hbm,
              idx_w, idx_p, bw, bp,
              sem_w0, sem_w1, sem_p0, sem_p1, sem_o0, sem_o1):
    wid = lax.axis_index("s")
    base = wid * B_PER_W
    pltpu.sync_copy(widx_hbm.at[pl.ds(base, B_PER_W)], idx_w)
    pltpu.sync_copy(pidx_hbm.at[pl.ds(base, B_PER_W)], idx_p)

    sems_w = (sem_w0, sem_w1)
    sems_p = (sem_p0, sem_p1)
    sems_o = (sem_o0, sem_o1)
    gath = [None, None]
    outd = [None, None]

    for c in range(NCHUNK + 1):
        k = c % 2
        if c < NCHUNK:
            if outd[k] is not None:
                outd[k].wait()
            gath[k] = (
                pltpu.async_copy(
                    word_hbm.at[idx_w.at[pl.ds(c * CHUNK, CHUNK)]],
                    bw.at[k], sems_w[k]),
                pltpu.async_copy(
                    pos_hbm.at[idx_p.at[pl.ds(c * CHUNK, CHUNK)]],
                    bp.at[k], sems_p[k]),
            )
        if c >= 1:
            kp = (c - 1) % 2
            gath[kp][0].wait()
            gath[kp][1].wait()

            @plsc.parallel_loop(0, CHUNK, step=1)
            def row_body(r, kp=kp):
                for j in range(COLS):
                    sl = (r, pl.ds(j * LANES, LANES))
                    plsc.addupdate(bw.at[kp].at[sl], bp.at[kp][sl])

            outd[kp] = pltpu.async_copy(
                bw.at[kp],
                out_hbm.at[pl.ds(base + (c - 1) * CHUNK, CHUNK)],
                sems_o[kp])
    for k in range(2):
        if outd[k] is not None:
            outd[k].wait()


@jax.jit
def _run(widx, pidx, word_table, pos_table):
    halves = []
    for h in range(2):
        mesh = plsc.VectorSubcoreMesh(
            core_axis_name="c", subcore_axis_name="s", num_cores=1)
        k = functools.partial(
            pl.kernel,
            mesh=mesh,
            out_type=jax.ShapeDtypeStruct((B_HALF, HIDDEN), jnp.float32),
            scratch_types=[
                pltpu.VMEM((B_PER_W,), jnp.int32),
                pltpu.VMEM((B_PER_W,), jnp.int32),
                pltpu.VMEM((2, CHUNK, HIDDEN), jnp.float32),
                pltpu.VMEM((2, CHUNK, HIDDEN), jnp.float32),
                pltpu.SemaphoreType.DMA,
                pltpu.SemaphoreType.DMA,
                pltpu.SemaphoreType.DMA,
                pltpu.SemaphoreType.DMA,
                pltpu.SemaphoreType.DMA,
                pltpu.SemaphoreType.DMA,
            ],
        )(_emb_body)
        halves.append(
            k(widx[h * B_HALF:(h + 1) * B_HALF],
              pidx[h * B_HALF:(h + 1) * B_HALF],
              word_table, pos_table))
    return jnp.concatenate(halves, axis=0)


def kernel(input_indices, position_indices, word_table, pos_table):
    widx = input_indices.reshape(-1).astype(jnp.int32)
    pidx = position_indices.reshape(-1).astype(jnp.int32)
    out = _run(widx, pidx, word_table, pos_table)
    return out.reshape(input_indices.shape + (HIDDEN,))


# speedup vs baseline: 1.4823x; 1.4823x over previous
import functools

import jax
import jax.numpy as jnp
from jax import lax
from jax.experimental import pallas as pl
from jax.experimental.pallas import tpu as pltpu
from jax.experimental.pallas import tpu_sc as plsc

HIDDEN = 768
B_TOTAL = 8192
B_HALF = B_TOTAL // 2
NS = 16
B_PER_W = B_HALF // NS
CHUNK = 32
NCHUNK = B_PER_W // CHUNK
LANES = 16
COLS = HIDDEN // LANES


def _emb_body(widx_hbm, pidx_hbm, word_hbm, pos_hbm, out0_hbm, out1_hbm,
              idx_w, idx_p, bw, bp,
              sem_w0, sem_w1, sem_p0, sem_p1, sem_o0, sem_o1):
    cid = lax.axis_index("c")
    sid = lax.axis_index("s")
    base = sid * B_PER_W
    gbase = cid * B_HALF + base
    pltpu.sync_copy(widx_hbm.at[pl.ds(gbase, B_PER_W)], idx_w)
    pltpu.sync_copy(pidx_hbm.at[pl.ds(gbase, B_PER_W)], idx_p)

    sems_w = (sem_w0, sem_w1)
    sems_p = (sem_p0, sem_p1)
    sems_o = (sem_o0, sem_o1)
    gath = [None, None]
    outd = [False, False]

    for c in range(NCHUNK + 1):
        k = c % 2
        if c < NCHUNK:
            if outd[k]:
                pltpu.make_async_copy(
                    bw.at[k], out0_hbm.at[pl.ds(base, CHUNK)],
                    sems_o[k]).wait()
                outd[k] = False
            gath[k] = (
                pltpu.async_copy(
                    word_hbm.at[idx_w.at[pl.ds(c * CHUNK, CHUNK)]],
                    bw.at[k], sems_w[k]),
                pltpu.async_copy(
                    pos_hbm.at[idx_p.at[pl.ds(c * CHUNK, CHUNK)]],
                    bp.at[k], sems_p[k]),
            )
        if c >= 1:
            cp_ = c - 1
            kp = cp_ % 2
            gath[kp][0].wait()
            gath[kp][1].wait()

            @plsc.parallel_loop(0, CHUNK, step=1)
            def row_body(r, kp=kp):
                for j in range(COLS):
                    sl = (r, pl.ds(j * LANES, LANES))
                    plsc.addupdate(bw.at[kp].at[sl], bp.at[kp][sl])

            dst = pl.ds(base + cp_ * CHUNK, CHUNK)

            @pl.when(cid == 0)
            def _(kp=kp, dst=dst):
                pltpu.async_copy(bw.at[kp], out0_hbm.at[dst], sems_o[kp])

            @pl.when(cid == 1)
            def _(kp=kp, dst=dst):
                pltpu.async_copy(bw.at[kp], out1_hbm.at[dst], sems_o[kp])

            outd[kp] = True
    for k in range(2):
        if outd[k]:
            pltpu.make_async_copy(
                bw.at[k], out0_hbm.at[pl.ds(base, CHUNK)], sems_o[k]).wait()


@jax.jit
def _run(widx, pidx, word_table, pos_table):
    mesh = plsc.VectorSubcoreMesh(core_axis_name="c", subcore_axis_name="s")
    k = functools.partial(
        pl.kernel,
        mesh=mesh,
        out_type=[
            jax.ShapeDtypeStruct((B_HALF, HIDDEN), jnp.float32),
            jax.ShapeDtypeStruct((B_HALF, HIDDEN), jnp.float32),
        ],
        scratch_types=[
            pltpu.VMEM((B_PER_W,), jnp.int32),
            pltpu.VMEM((B_PER_W,), jnp.int32),
            pltpu.VMEM((2, CHUNK, HIDDEN), jnp.float32),
            pltpu.VMEM((2, CHUNK, HIDDEN), jnp.float32),
            pltpu.SemaphoreType.DMA,
            pltpu.SemaphoreType.DMA,
            pltpu.SemaphoreType.DMA,
            pltpu.SemaphoreType.DMA,
            pltpu.SemaphoreType.DMA,
            pltpu.SemaphoreType.DMA,
        ],
    )(_emb_body)
    o0, o1 = k(widx, pidx, word_table, pos_table)
    return jnp.concatenate([o0, o1], axis=0)


def kernel(input_indices, position_indices, word_table, pos_table):
    widx = input_indices.reshape(-1).astype(jnp.int32)
    pidx = position_indices.reshape(-1).astype(jnp.int32)
    out = _run(widx, pidx, word_table, pos_table)
    return out.reshape(input_indices.shape + (HIDDEN,))


# speedup vs baseline: 1.9415x; 1.3098x over previous
import functools

import jax
import jax.numpy as jnp
from jax import lax
from jax.experimental import pallas as pl
from jax.experimental.pallas import tpu as pltpu
from jax.experimental.pallas import tpu_sc as plsc

HIDDEN = 768
B_TOTAL = 8192
NW = 32
B_PER_W = B_TOTAL // NW
CHUNK = 32
NCHUNK = B_PER_W // CHUNK
LANES = 16
COLS = HIDDEN // LANES


def _emb_body(widx_hbm, pidx_hbm, word_hbm, pos_hbm, out_hbm,
              idx_w, idx_p, bw, bp,
              sem_w0, sem_w1, sem_p0, sem_p1, sem_o0, sem_o1):
    wid = lax.axis_index("s") * 2 + lax.axis_index("c")
    base = wid * B_PER_W
    pltpu.sync_copy(widx_hbm.at[pl.ds(base, B_PER_W)], idx_w)
    pltpu.sync_copy(pidx_hbm.at[pl.ds(base, B_PER_W)], idx_p)

    sems_w = (sem_w0, sem_w1)
    sems_p = (sem_p0, sem_p1)
    sems_o = (sem_o0, sem_o1)
    gath = [None, None]
    outd = [None, None]

    for c in range(NCHUNK + 1):
        k = c % 2
        if c < NCHUNK:
            if outd[k] is not None:
                outd[k].wait()
            gath[k] = (
                pltpu.async_copy(
                    word_hbm.at[idx_w.at[pl.ds(c * CHUNK, CHUNK)]],
                    bw.at[k], sems_w[k]),
                pltpu.async_copy(
                    pos_hbm.at[idx_p.at[pl.ds(c * CHUNK, CHUNK)]],
                    bp.at[k], sems_p[k]),
            )
        if c >= 1:
            kp = (c - 1) % 2
            gath[kp][0].wait()
            gath[kp][1].wait()

            @plsc.parallel_loop(0, CHUNK, step=1)
            def row_body(r, kp=kp):
                for j in range(COLS):
                    sl = (r, pl.ds(j * LANES, LANES))
                    plsc.addupdate(bw.at[kp].at[sl], bp.at[kp][sl])

            outd[kp] = pltpu.async_copy(
                bw.at[kp],
                out_hbm.at[pl.ds(base + (c - 1) * CHUNK, CHUNK)],
                sems_o[kp])
    for k in range(2):
        if outd[k] is not None:
            outd[k].wait()


@jax.jit
def _run(widx, pidx, word_table, pos_table):
    mesh = plsc.VectorSubcoreMesh(core_axis_name="c", subcore_axis_name="s")
    k = functools.partial(
        pl.kernel,
        mesh=mesh,
        out_type=jax.ShapeDtypeStruct((B_TOTAL, HIDDEN), jnp.float32),
        scratch_types=[
            pltpu.VMEM((B_PER_W,), jnp.int32),
            pltpu.VMEM((B_PER_W,), jnp.int32),
            pltpu.VMEM((2, CHUNK, HIDDEN), jnp.float32),
            pltpu.VMEM((2, CHUNK, HIDDEN), jnp.float32),
            pltpu.SemaphoreType.DMA,
            pltpu.SemaphoreType.DMA,
            pltpu.SemaphoreType.DMA,
            pltpu.SemaphoreType.DMA,
            pltpu.SemaphoreType.DMA,
            pltpu.SemaphoreType.DMA,
        ],
    )(_emb_body)
    return k(widx, pidx, word_table, pos_table)


def kernel(input_indices, position_indices, word_table, pos_table):
    widx = input_indices.reshape(-1).astype(jnp.int32)
    pidx = position_indices.reshape(-1).astype(jnp.int32)
    out = _run(widx, pidx, word_table, pos_table)
    return out.reshape(input_indices.shape + (HIDDEN,))
